# initial kernel scaffold (unmeasured)
import jax
import jax.numpy as jnp
from jax import lax
from jax.experimental import pallas as pl
from jax.experimental.pallas import tpu as pltpu

N_DEV = 16
E_PER = 4
N_EXP = 64
CAP = 409


def _incl_cumsum_rows(a):
    n = a.shape[0]
    s = a
    k = 1
    while k < n:
        z = jnp.zeros((k, a.shape[1]), a.dtype)
        s = s + jnp.concatenate([z, s[:-k, :]], axis=0)
        k *= 2
    return s


def kernel(x, router_W, route_idx, expert_W):
    T, D = x.shape
    H = expert_W.shape[-1]

    def body(x_ref, ridx_ref, ew_ref, out_ref,
             table_ref, wcomm_ref, hsend, hrecv, wsend, wrecv, credit_sem):
        me = lax.axis_index("i")
        left = jnp.mod(me - 1, N_DEV)
        right = jnp.mod(me + 1, N_DEV)

        barrier_sem = pltpu.get_barrier_semaphore()
        for nbr in (left, right):
            pl.semaphore_signal(barrier_sem, inc=1, device_id=(nbr,),
                                device_id_type=pl.DeviceIdType.MESH)
        pl.semaphore_wait(barrier_sem, 2)

        ridx = ridx_ref[:, :]
        lanes = lax.broadcasted_iota(jnp.int32, (T, N_EXP), 1)
        eq = (ridx == lanes).astype(jnp.int32)
        hist = jnp.sum(eq, axis=0, keepdims=True)
        table_ref[pl.ds(me, 1), :] = hist

        for h in range(N_DEV - 1):
            src_row = jnp.mod(me - h, N_DEV)
            rdma = pltpu.make_async_remote_copy(
                src_ref=table_ref.at[pl.ds(src_row, 1)],
                dst_ref=table_ref.at[pl.ds(src_row, 1)],
                send_sem=hsend.at[h],
                recv_sem=hrecv.at[h],
                device_id=(right,),
                device_id_type=pl.DeviceIdType.MESH,
            )
            rdma.start()
            rdma.wait()

        rows = lax.broadcasted_iota(jnp.int32, (N_DEV, N_EXP), 0)
        prev = jnp.sum(jnp.where(rows < me, table_ref[:, :], 0),
                       axis=0, keepdims=True)
        lcum = _incl_cumsum_rows(eq) - eq
        rank = jnp.sum((prev + lcum) * eq, axis=1, keepdims=True)
        keep = rank < CAP

        xv = x_ref[:, :]
        out_ref[:, :] = jnp.zeros((T, H), jnp.float32)

        def accum_block(origin, get_w):
            for j in range(E_PER):
                e = origin * E_PER + j
                m = jnp.where(keep & (ridx == e), 1.0, 0.0)
                out_ref[:, :] += jnp.dot(
                    xv * m, get_w(j), preferred_element_type=jnp.float32)

        accum_block(me, lambda j: ew_ref[j])

        for h in range(N_DEV - 1):
            if h >= 2:
                pl.semaphore_wait(credit_sem, 1)
            src = ew_ref if h == 0 else wcomm_ref.at[(h - 1) % 2]
            rdma = pltpu.make_async_remote_copy(
                src_ref=src,
                dst_ref=wcomm_ref.at[h % 2],
                send_sem=wsend.at[h % 2],
                recv_sem=wrecv.at[h % 2],
                device_id=(right,),
                device_id_type=pl.DeviceIdType.MESH,
            )
            rdma.start()
            rdma.wait()
            origin = jnp.mod(me - h - 1, N_DEV)
            accum_block(origin, lambda j: wcomm_ref[h % 2, j])
            if h <= N_DEV - 4:
                pl.semaphore_signal(credit_sem, inc=1, device_id=(left,),
                                    device_id_type=pl.DeviceIdType.MESH)

    return pl.pallas_call(
        body,
        out_shape=jax.ShapeDtypeStruct((T, H), jnp.float32),
        in_specs=[pl.BlockSpec(memory_space=pltpu.VMEM)] * 3,
        out_specs=pl.BlockSpec(memory_space=pltpu.VMEM),
        scratch_shapes=[
            pltpu.VMEM((N_DEV, N_EXP), jnp.int32),
            pltpu.VMEM((2, E_PER, D, H), jnp.float32),
            pltpu.SemaphoreType.DMA((N_DEV - 1,)),
            pltpu.SemaphoreType.DMA((N_DEV - 1,)),
            pltpu.SemaphoreType.DMA((2,)),
            pltpu.SemaphoreType.DMA((2,)),
            pltpu.SemaphoreType.REGULAR,
        ],
        compiler_params=pltpu.CompilerParams(collective_id=0),
    )(x, route_idx, expert_W)


# baseline (device time: 1617754 ns/iter reference)
import functools

import jax
import jax.numpy as jnp
from jax import lax
from jax.experimental import pallas as pl
from jax.experimental.pallas import tpu as pltpu

N_DEV = 16
E_PER = 4
N_EXP = 64
CAP = 409


def _incl_cumsum_rows(a):
    n = a.shape[0]
    s = a
    k = 1
    while k < n:
        z = jnp.zeros((k, a.shape[1]), a.dtype)
        s = s + jnp.concatenate([z, s[:-k, :]], axis=0)
        k *= 2
    return s


def kernel(x, router_W, route_idx, expert_W):
    T, D = x.shape
    H = expert_W.shape[-1]

    def body(x_ref, ridx_ref, ew_ref, out_ref,
             table_ref, wcomm_ref, hsend, hrecv, wsend, wrecv,
             prime_sem, hcredit, wcredit):
        me = lax.axis_index("i")
        left = jnp.mod(me - 1, N_DEV)
        right = jnp.mod(me + 1, N_DEV)

        barrier_sem = pltpu.get_barrier_semaphore()
        for nbr in (left, right):
            pl.semaphore_signal(barrier_sem, inc=1, device_id=(nbr,),
                                device_id_type=pl.DeviceIdType.MESH)
        pl.semaphore_wait(barrier_sem, 2)

        prime = pltpu.make_async_copy(ew_ref, wcomm_ref.at[0], prime_sem)
        prime.start()
        prime.wait()

        ridx = ridx_ref[:, :]
        lanes = lax.broadcasted_iota(jnp.int32, (T, N_EXP), 1)
        eq = (ridx == lanes).astype(jnp.int32)
        hist = jnp.sum(eq, axis=0, keepdims=True)
        table_ref[pl.ds(me, 1), :] = hist

        def hist_hop(h, carry):
            @pl.when(h >= 1)
            def _():
                pl.semaphore_wait(hcredit, 1)
            src_row = jnp.mod(me - h, N_DEV)
            rdma = pltpu.make_async_remote_copy(
                src_ref=table_ref.at[pl.ds(src_row, 1)],
                dst_ref=table_ref.at[pl.ds(src_row, 1)],
                send_sem=hsend,
                recv_sem=hrecv,
                device_id=(right,),
                device_id_type=pl.DeviceIdType.MESH,
            )
            rdma.start()
            rdma.wait()

            @pl.when(h <= N_DEV - 3)
            def _():
                pl.semaphore_signal(hcredit, inc=1, device_id=(left,),
                                    device_id_type=pl.DeviceIdType.MESH)
            return carry

        lax.fori_loop(0, N_DEV - 1, hist_hop, 0)

        rows = lax.broadcasted_iota(jnp.int32, (N_DEV, N_EXP), 0)
        prev = jnp.sum(jnp.where(rows < me, table_ref[:, :], 0),
                       axis=0, keepdims=True)
        lcum = _incl_cumsum_rows(eq) - eq
        rank = jnp.sum((prev + lcum) * eq, axis=1, keepdims=True)
        keep = rank < CAP

        xv = x_ref[:, :]
        out_ref[:, :] = jnp.zeros((T, H), jnp.float32)

        def wstep(s, carry):
            origin = jnp.mod(me - s, N_DEV)

            def do_slot(slot):
                nslot = 1 - slot

                @pl.when(s > 0)
                def _():
                    recv = pltpu.make_async_remote_copy(
                        src_ref=wcomm_ref.at[slot],
                        dst_ref=wcomm_ref.at[slot],
                        send_sem=wsend.at[slot],
                        recv_sem=wrecv.at[slot],
                        device_id=(right,),
                        device_id_type=pl.DeviceIdType.MESH,
                    )
                    recv.wait_recv()

                @pl.when(s < N_DEV - 1)
                def _():
                    @pl.when(s >= 1)
                    def _():
                        pl.semaphore_wait(wcredit, 1)
                    snd = pltpu.make_async_remote_copy(
                        src_ref=wcomm_ref.at[slot],
                        dst_ref=wcomm_ref.at[nslot],
                        send_sem=wsend.at[slot],
                        recv_sem=wrecv.at[nslot],
                        device_id=(right,),
                        device_id_type=pl.DeviceIdType.MESH,
                    )
                    snd.start()
                    snd.wait_send()

                for j in range(E_PER):
                    e = origin * E_PER + j
                    m = jnp.where(keep & (ridx == e), 1.0, 0.0)
                    out_ref[:, :] += jnp.dot(
                        xv * m, wcomm_ref[slot, j],
                        preferred_element_type=jnp.float32)

            @pl.when(s % 2 == 0)
            def _():
                do_slot(0)

            @pl.when(s % 2 == 1)
            def _():
                do_slot(1)

            @pl.when(s <= N_DEV - 3)
            def _():
                pl.semaphore_signal(wcredit, inc=1, device_id=(left,),
                                    device_id_type=pl.DeviceIdType.MESH)
            return carry

        lax.fori_loop(0, N_DEV, wstep, 0)

    return pl.pallas_call(
        body,
        out_shape=jax.ShapeDtypeStruct((T, H), jnp.float32),
        in_specs=[pl.BlockSpec(memory_space=pltpu.VMEM)] * 3,
        out_specs=pl.BlockSpec(memory_space=pltpu.VMEM),
        scratch_shapes=[
            pltpu.VMEM((N_DEV, N_EXP), jnp.int32),
            pltpu.VMEM((2, E_PER, D, H), jnp.float32),
            pltpu.SemaphoreType.DMA,
            pltpu.SemaphoreType.DMA,
            pltpu.SemaphoreType.DMA((2,)),
            pltpu.SemaphoreType.DMA((2,)),
            pltpu.SemaphoreType.DMA,
            pltpu.SemaphoreType.REGULAR,
            pltpu.SemaphoreType.REGULAR,
        ],
        compiler_params=pltpu.CompilerParams(
            collective_id=0,
            vmem_limit_bytes=100 * 1024 * 1024,
        ),
    )(x, route_idx, expert_W)
